# Initial kernel scaffold; baseline (speedup 1.0000x reference)
import jax
import jax.numpy as jnp
from jax.experimental import pallas as pl


def kernel(xyz, points, affine_alpha, affine_beta):
    raise NotImplementedError("write your pallas kernel here")


# speedup vs baseline: 1.5627x; 1.5627x over previous
import functools

import jax
import jax.numpy as jnp
from jax.experimental import pallas as pl
from jax.experimental.pallas import tpu as pltpu

_B, _N, _C = 8, 8192, 3
_D = 128
_S = 2048
_K = 32


_FW = 128


def _fps_body(xyz_ref, idx_ref, nxyz_ref, dist_ref):
    x = xyz_ref[0]
    y = xyz_ref[1]
    z = xyz_ref[2]
    lane = jax.lax.broadcasted_iota(jnp.int32, (_B, _N), 1)
    col = jax.lax.broadcasted_iota(jnp.int32, (_B, _FW), 1)
    dist_ref[...] = jnp.full((_B, _N), 1e10, jnp.float32)

    def inner(i, carry):
        far, ai, ax, ay, az = carry
        sel = lane == far
        cx = jnp.sum(jnp.where(sel, x, 0.0), axis=1, keepdims=True)
        cy = jnp.sum(jnp.where(sel, y, 0.0), axis=1, keepdims=True)
        cz = jnp.sum(jnp.where(sel, z, 0.0), axis=1, keepdims=True)
        here = col == i
        ai = jnp.where(here, far, ai)
        ax = jnp.where(here, cx, ax)
        ay = jnp.where(here, cy, ay)
        az = jnp.where(here, cz, az)
        dx = x - cx
        dy = y - cy
        dz = z - cz
        d = (dx * dx + dy * dy) + dz * dz
        dist = jnp.minimum(dist_ref[...], d)
        dist_ref[...] = dist
        mx = jnp.max(dist, axis=1, keepdims=True)
        cand = jnp.where(dist == mx, lane, _N)
        far = jnp.min(cand, axis=1, keepdims=True)
        return far, ai, ax, ay, az

    def outer(tb, far):
        zi = jnp.zeros((_B, _FW), jnp.int32)
        zf = jnp.zeros((_B, _FW), jnp.float32)
        far, ai, ax, ay, az = jax.lax.fori_loop(
            0, _FW, inner, (far, zi, zf, zf, zf)
        )
        base = pl.multiple_of(tb * _FW, _FW)
        idx_ref[:, pl.ds(base, _FW)] = ai
        nxyz_ref[0, :, pl.ds(base, _FW)] = ax
        nxyz_ref[1, :, pl.ds(base, _FW)] = ay
        nxyz_ref[2, :, pl.ds(base, _FW)] = az
        return far

    jax.lax.fori_loop(0, _S // _FW, outer, jnp.zeros((_B, 1), jnp.int32))


@functools.partial(jax.jit, static_argnames=("interpret",))
def _fps(xyz_t, interpret=False):
    return pl.pallas_call(
        _fps_body,
        out_shape=[
            jax.ShapeDtypeStruct((_B, _S), jnp.int32),
            jax.ShapeDtypeStruct((3, _B, _S), jnp.float32),
        ],
        scratch_shapes=[pltpu.VMEM((_B, _N), jnp.float32)],
        interpret=interpret,
    )(xyz_t)


def _square_distance(src, dst):
    dist = -2.0 * jnp.matmul(src, jnp.swapaxes(dst, 1, 2))
    dist = dist + jnp.sum(src**2, axis=-1)[:, :, None]
    dist = dist + jnp.sum(dst**2, axis=-1)[:, None, :]
    return dist


def _index_points(points, idx):
    return jax.vmap(lambda p, i: p[i])(points, idx)


def kernel(xyz, points, affine_alpha, affine_beta):
    xyz_t = jnp.transpose(xyz, (2, 0, 1))
    fps_idx, nxyz = _fps(xyz_t)
    new_xyz = jnp.transpose(nxyz, (1, 2, 0))
    new_points = _index_points(points, fps_idx)

    sqrdists = _square_distance(new_xyz, xyz)
    _, idx = jax.lax.top_k(-sqrdists, _K)
    grouped_xyz = _index_points(xyz, idx)
    grouped_points = _index_points(points, idx)
    grouped_points = jnp.concatenate([grouped_points, grouped_xyz], axis=-1)
    mean = jnp.mean(grouped_points, axis=2, keepdims=True)
    centered = grouped_points - mean
    flat = centered.reshape(_B, -1)
    n = flat.shape[-1]
    fmean = jnp.mean(flat, axis=-1, keepdims=True)
    var = jnp.sum((flat - fmean) ** 2, axis=-1, keepdims=True) / (n - 1)
    std = jnp.sqrt(var)[:, :, None, None]
    grouped_points = centered / (std + 1e-05)
    grouped_points = affine_alpha * grouped_points + affine_beta
    new_points_out = jnp.concatenate(
        [grouped_points, jnp.tile(new_points.reshape(_B, _S, 1, -1), (1, 1, _K, 1))],
        axis=-1,
    )
    return (new_xyz, new_points_out)


# speedup vs baseline: 8.1621x; 5.2230x over previous
import functools

import jax
import jax.numpy as jnp
from jax.experimental import pallas as pl
from jax.experimental.pallas import tpu as pltpu

_B, _N, _C = 8, 8192, 3
_D = 128
_S = 2048
_K = 32


_FW = 128


def _fps_body(xyz_ref, idx_ref, nxyz_ref, dist_ref):
    x = xyz_ref[0]
    y = xyz_ref[1]
    z = xyz_ref[2]
    lane = jax.lax.broadcasted_iota(jnp.int32, (_B, _N), 1)
    col = jax.lax.broadcasted_iota(jnp.int32, (_B, _FW), 1)
    dist_ref[...] = jnp.full((_B, _N), 1e10, jnp.float32)

    def inner(i, carry):
        far, ai, ax, ay, az = carry
        sel = lane == far
        cx = jnp.sum(jnp.where(sel, x, 0.0), axis=1, keepdims=True)
        cy = jnp.sum(jnp.where(sel, y, 0.0), axis=1, keepdims=True)
        cz = jnp.sum(jnp.where(sel, z, 0.0), axis=1, keepdims=True)
        here = col == i
        ai = jnp.where(here, far, ai)
        ax = jnp.where(here, cx, ax)
        ay = jnp.where(here, cy, ay)
        az = jnp.where(here, cz, az)
        dx = x - cx
        dy = y - cy
        dz = z - cz
        d = (dx * dx + dy * dy) + dz * dz
        dist = jnp.minimum(dist_ref[...], d)
        dist_ref[...] = dist
        mx = jnp.max(dist, axis=1, keepdims=True)
        cand = jnp.where(dist == mx, lane, _N)
        far = jnp.min(cand, axis=1, keepdims=True)
        return far, ai, ax, ay, az

    def outer(tb, far):
        zi = jnp.zeros((_B, _FW), jnp.int32)
        zf = jnp.zeros((_B, _FW), jnp.float32)
        far, ai, ax, ay, az = jax.lax.fori_loop(
            0, _FW, inner, (far, zi, zf, zf, zf)
        )
        base = pl.multiple_of(tb * _FW, _FW)
        idx_ref[:, pl.ds(base, _FW)] = ai
        nxyz_ref[0, :, pl.ds(base, _FW)] = ax
        nxyz_ref[1, :, pl.ds(base, _FW)] = ay
        nxyz_ref[2, :, pl.ds(base, _FW)] = az
        return far

    jax.lax.fori_loop(0, _S // _FW, outer, jnp.zeros((_B, 1), jnp.int32))


@functools.partial(jax.jit, static_argnames=("interpret",))
def _fps(xyz_t, interpret=False):
    return pl.pallas_call(
        _fps_body,
        out_shape=[
            jax.ShapeDtypeStruct((_B, _S), jnp.int32),
            jax.ShapeDtypeStruct((3, _B, _S), jnp.float32),
        ],
        scratch_shapes=[pltpu.VMEM((_B, _N), jnp.float32)],
        interpret=interpret,
    )(xyz_t)


def _index_points(points, idx):
    return jax.vmap(lambda p, i: p[i])(points, idx)


_Q = 256


def _knn_body(nxyz_ref, xyzt_ref, idx_ref):
    b = pl.program_id(0)
    qx = nxyz_ref[0, :, 0:1]
    qy = nxyz_ref[0, :, 1:2]
    qz = nxyz_ref[0, :, 2:3]
    px = xyzt_ref[0, 0]
    py = xyzt_ref[0, 1]
    pz = xyzt_ref[0, 2]
    qn = (qx * qx + qy * qy) + qz * qz
    pn = (px * px + py * py) + pz * pz
    qxb = qx.astype(jnp.bfloat16).astype(jnp.float32)
    qyb = qy.astype(jnp.bfloat16).astype(jnp.float32)
    qzb = qz.astype(jnp.bfloat16).astype(jnp.float32)
    pxb = px.astype(jnp.bfloat16).astype(jnp.float32)
    pyb = py.astype(jnp.bfloat16).astype(jnp.float32)
    pzb = pz.astype(jnp.bfloat16).astype(jnp.float32)
    prod = (qxb * pxb + qyb * pyb) + qzb * pzb
    dist = ((-2.0 * prod) + qn) + pn
    lane = jax.lax.broadcasted_iota(jnp.int32, (_Q, _N), 1)
    col = jax.lax.broadcasted_iota(jnp.int32, (_Q, _K), 1)
    acc = jnp.zeros((_Q, _K), jnp.int32)
    for k in range(_K):
        m = jnp.min(dist, axis=1, keepdims=True)
        cand = jnp.min(jnp.where(dist == m, lane, _N), axis=1, keepdims=True)
        acc = jnp.where(col == k, cand, acc)
        dist = jnp.where(lane == cand, jnp.float32(1e30), dist)
    idx_ref[0] = acc + b * _N


@functools.partial(jax.jit, static_argnames=("interpret",))
def _knn(new_xyz, xyz_t, interpret=False):
    return pl.pallas_call(
        _knn_body,
        grid=(_B, _S // _Q),
        in_specs=[
            pl.BlockSpec((1, _Q, 3), lambda b, s: (b, s, 0)),
            pl.BlockSpec((1, 3, 1, _N), lambda b, s: (b, 0, 0, 0)),
        ],
        out_specs=pl.BlockSpec((1, _Q, _K), lambda b, s: (b, s, 0)),
        out_shape=jax.ShapeDtypeStruct((_B, _S, _K), jnp.int32),
        interpret=interpret,
    )(new_xyz, xyz_t)


def kernel(xyz, points, affine_alpha, affine_beta):
    xyz_t = jnp.transpose(xyz, (2, 0, 1))
    fps_idx, nxyz = _fps(xyz_t)
    new_xyz = jnp.transpose(nxyz, (1, 2, 0))
    new_points = _index_points(points, fps_idx)

    xyz_q = jnp.transpose(xyz, (0, 2, 1))[:, :, None, :]
    idx_g = _knn(new_xyz, xyz_q)
    xyz_flat = xyz.reshape(_B * _N, _C)
    pts_flat = points.reshape(_B * _N, _D)
    grouped_xyz = xyz_flat[idx_g]
    grouped_points = pts_flat[idx_g]
    grouped_points = jnp.concatenate([grouped_points, grouped_xyz], axis=-1)
    mean = jnp.mean(grouped_points, axis=2, keepdims=True)
    centered = grouped_points - mean
    flat = centered.reshape(_B, -1)
    n = flat.shape[-1]
    fmean = jnp.mean(flat, axis=-1, keepdims=True)
    var = jnp.sum((flat - fmean) ** 2, axis=-1, keepdims=True) / (n - 1)
    std = jnp.sqrt(var)[:, :, None, None]
    grouped_points = centered / (std + 1e-05)
    grouped_points = affine_alpha * grouped_points + affine_beta
    new_points_out = jnp.concatenate(
        [grouped_points, jnp.tile(new_points.reshape(_B, _S, 1, -1), (1, 1, _K, 1))],
        axis=-1,
    )
    return (new_xyz, new_points_out)


# speedup vs baseline: 10.2547x; 1.2564x over previous
import functools

import jax
import jax.numpy as jnp
from jax.experimental import pallas as pl
from jax.experimental.pallas import tpu as pltpu
from jax.experimental.pallas import tpu_sc as plsc

_B, _N, _C = 8, 8192, 3
_D = 128
_S = 2048
_K = 32


_FW = 128


def _fps_body(xyz_ref, idx_ref, nxyz_ref, dist_ref):
    x = xyz_ref[0]
    y = xyz_ref[1]
    z = xyz_ref[2]
    lane = jax.lax.broadcasted_iota(jnp.int32, (_B, _N), 1)
    col = jax.lax.broadcasted_iota(jnp.int32, (_B, _FW), 1)
    dist_ref[...] = jnp.full((_B, _N), 1e10, jnp.float32)

    def inner(i, carry):
        far, ai, ax, ay, az = carry
        sel = lane == far
        cx = jnp.sum(jnp.where(sel, x, 0.0), axis=1, keepdims=True)
        cy = jnp.sum(jnp.where(sel, y, 0.0), axis=1, keepdims=True)
        cz = jnp.sum(jnp.where(sel, z, 0.0), axis=1, keepdims=True)
        here = col == i
        ai = jnp.where(here, far, ai)
        ax = jnp.where(here, cx, ax)
        ay = jnp.where(here, cy, ay)
        az = jnp.where(here, cz, az)
        dx = x - cx
        dy = y - cy
        dz = z - cz
        d = (dx * dx + dy * dy) + dz * dz
        dist = jnp.minimum(dist_ref[...], d)
        dist_ref[...] = dist
        mx = jnp.max(dist, axis=1, keepdims=True)
        cand = jnp.where(dist == mx, lane, _N)
        far = jnp.min(cand, axis=1, keepdims=True)
        return far, ai, ax, ay, az

    def outer(tb, far):
        zi = jnp.zeros((_B, _FW), jnp.int32)
        zf = jnp.zeros((_B, _FW), jnp.float32)
        far, ai, ax, ay, az = jax.lax.fori_loop(
            0, _FW, inner, (far, zi, zf, zf, zf)
        )
        base = pl.multiple_of(tb * _FW, _FW)
        idx_ref[:, pl.ds(base, _FW)] = ai
        nxyz_ref[0, :, pl.ds(base, _FW)] = ax
        nxyz_ref[1, :, pl.ds(base, _FW)] = ay
        nxyz_ref[2, :, pl.ds(base, _FW)] = az
        return far

    jax.lax.fori_loop(0, _S // _FW, outer, jnp.zeros((_B, 1), jnp.int32))


@functools.partial(jax.jit, static_argnames=("interpret",))
def _fps(xyz_t, interpret=False):
    return pl.pallas_call(
        _fps_body,
        out_shape=[
            jax.ShapeDtypeStruct((_B, _S), jnp.int32),
            jax.ShapeDtypeStruct((3, _B, _S), jnp.float32),
        ],
        scratch_shapes=[pltpu.VMEM((_B, _N), jnp.float32)],
        interpret=interpret,
    )(xyz_t)


def _index_points(points, idx):
    return jax.vmap(lambda p, i: p[i])(points, idx)


_Q = 256


def _knn_body(nxyz_ref, xyzt_ref, idx_ref):
    b = pl.program_id(0)
    qx = nxyz_ref[0, :, 0:1]
    qy = nxyz_ref[0, :, 1:2]
    qz = nxyz_ref[0, :, 2:3]
    px = xyzt_ref[0, 0]
    py = xyzt_ref[0, 1]
    pz = xyzt_ref[0, 2]
    qn = (qx * qx + qy * qy) + qz * qz
    pn = (px * px + py * py) + pz * pz
    qxb = qx.astype(jnp.bfloat16).astype(jnp.float32)
    qyb = qy.astype(jnp.bfloat16).astype(jnp.float32)
    qzb = qz.astype(jnp.bfloat16).astype(jnp.float32)
    pxb = px.astype(jnp.bfloat16).astype(jnp.float32)
    pyb = py.astype(jnp.bfloat16).astype(jnp.float32)
    pzb = pz.astype(jnp.bfloat16).astype(jnp.float32)
    prod = (qxb * pxb + qyb * pyb) + qzb * pzb
    dist = ((-2.0 * prod) + qn) + pn
    lane = jax.lax.broadcasted_iota(jnp.int32, (_Q, _N), 1)
    col = jax.lax.broadcasted_iota(jnp.int32, (_Q, _K), 1)
    acc = jnp.zeros((_Q, _K), jnp.int32)
    for k in range(_K):
        m = jnp.min(dist, axis=1, keepdims=True)
        cand = jnp.min(jnp.where(dist == m, lane, _N), axis=1, keepdims=True)
        acc = jnp.where(col == k, cand, acc)
        dist = jnp.where(lane == cand, jnp.float32(1e30), dist)
    idx_ref[0] = acc + b * _N


@functools.partial(jax.jit, static_argnames=("interpret",))
def _knn(new_xyz, xyz_t, interpret=False):
    return pl.pallas_call(
        _knn_body,
        grid=(_B, _S // _Q),
        in_specs=[
            pl.BlockSpec((1, _Q, 3), lambda b, s: (b, s, 0)),
            pl.BlockSpec((1, 3, 1, _N), lambda b, s: (b, 0, 0, 0)),
        ],
        out_specs=pl.BlockSpec((1, _Q, _K), lambda b, s: (b, s, 0)),
        out_shape=jax.ShapeDtypeStruct((_B, _S, _K), jnp.int32),
        interpret=interpret,
    )(new_xyz, xyz_t)


_FD = 131
_GW = 256
_GW3 = 4096


def _sc_gather_rows(table, idxs):
    info = plsc.get_sparse_core_info()
    nw = info.num_cores * info.num_subcores
    tot = idxs.shape[0]
    b_per_w = tot // nw
    nchunks = b_per_w // _GW
    mesh = plsc.VectorSubcoreMesh(core_axis_name="c", subcore_axis_name="s")

    @functools.partial(
        pl.kernel,
        mesh=mesh,
        out_type=jax.ShapeDtypeStruct((tot, _D), jnp.float32),
        scratch_types=[
            pltpu.VMEM((_GW,), jnp.int32),
            pltpu.VMEM((_GW, _D), jnp.float32),
            pltpu.SemaphoreType.DMA,
        ],
    )
    def k(table_hbm, idx_hbm, out_hbm, idx_v, rows_v, sem):
        wid = jax.lax.axis_index("s") * info.num_cores + jax.lax.axis_index("c")
        base = wid * b_per_w

        def body(c, carry):
            off = base + c * _GW
            pltpu.sync_copy(idx_hbm.at[pl.ds(off, _GW)], idx_v)
            pltpu.async_copy(table_hbm.at[idx_v], rows_v, sem).wait()
            pltpu.sync_copy(rows_v, out_hbm.at[pl.ds(off, _GW)])
            return carry

        jax.lax.fori_loop(0, nchunks, body, 0)

    return k(table, idxs)


def _sc_gather_words(table, idxs):
    info = plsc.get_sparse_core_info()
    nw = info.num_cores * info.num_subcores
    tot = idxs.shape[0]
    b_per_w = tot // nw
    nchunks = b_per_w // _GW3
    mesh = plsc.VectorSubcoreMesh(core_axis_name="c", subcore_axis_name="s")

    @functools.partial(
        pl.kernel,
        mesh=mesh,
        out_type=jax.ShapeDtypeStruct((tot,), jnp.float32),
        scratch_types=[
            pltpu.VMEM((_GW3,), jnp.int32),
            pltpu.VMEM((_GW3,), jnp.float32),
            pltpu.SemaphoreType.DMA,
        ],
    )
    def k(table_hbm, idx_hbm, out_hbm, idx_v, w_v, sem):
        wid = jax.lax.axis_index("s") * info.num_cores + jax.lax.axis_index("c")
        base = wid * b_per_w

        def body(c, carry):
            off = base + c * _GW3
            pltpu.sync_copy(idx_hbm.at[pl.ds(off, _GW3)], idx_v)
            pltpu.async_copy(table_hbm.at[idx_v], w_v, sem).wait()
            pltpu.sync_copy(w_v, out_hbm.at[pl.ds(off, _GW3)])
            return carry

        jax.lax.fori_loop(0, nchunks, body, 0)

    return k(table, idxs)


_T = 128


def _stats_body(gp_ref, gx_ref, out_ref):
    b = pl.program_id(0)
    st = pl.program_id(1)
    gp = gp_ref[0]
    gx = gx_ref[0]
    cp = gp - jnp.sum(gp, axis=1, keepdims=True) * jnp.float32(1.0 / _K)
    cx = gx - jnp.sum(gx, axis=1, keepdims=True) * jnp.float32(1.0 / _K)
    s1 = jnp.sum(cp) + jnp.sum(cx)
    s2 = jnp.sum(cp * cp) + jnp.sum(cx * cx)
    lane = jax.lax.broadcasted_iota(jnp.int32, (_B, 128), 1)
    row = jax.lax.broadcasted_iota(jnp.int32, (_B, 128), 0)
    contrib = jnp.where(
        (row == b) & (lane == 0),
        s1,
        jnp.where((row == b) & (lane == 1), s2, jnp.float32(0.0)),
    )

    @pl.when((b == 0) & (st == 0))
    def _():
        out_ref[...] = jnp.zeros((_B, 128), jnp.float32)

    out_ref[...] += contrib


@jax.jit
def _stats(gp, gx):
    return pl.pallas_call(
        _stats_body,
        grid=(_B, _S // _T),
        in_specs=[
            pl.BlockSpec((1, _T, _K, _D), lambda b, s: (b, s, 0, 0)),
            pl.BlockSpec((1, _T, _K, 3), lambda b, s: (b, s, 0, 0)),
        ],
        out_specs=pl.BlockSpec((_B, 128), lambda b, s: (0, 0)),
        out_shape=jax.ShapeDtypeStruct((_B, 128), jnp.float32),
    )(gp, gx)


def _assemble_body(gp_ref, gx_ref, np_ref, std_ref, ab_ref, out_ref):
    b = pl.program_id(0)
    gp = gp_ref[0]
    gx = gx_ref[0]
    cp = gp - jnp.sum(gp, axis=1, keepdims=True) * jnp.float32(1.0 / _K)
    cx = gx - jnp.sum(gx, axis=1, keepdims=True) * jnp.float32(1.0 / _K)
    denom = std_ref[b, 0] + jnp.float32(1e-05)
    a_p = ab_ref[0:1, :_D][None]
    a_x = ab_ref[0:1, _D:_FD][None]
    b_p = ab_ref[1:2, :_D][None]
    b_x = ab_ref[1:2, _D:_FD][None]
    y_p = a_p * (cp / denom) + b_p
    y_x = a_x * (cx / denom) + b_x
    npt = np_ref[0]
    tiled = jnp.broadcast_to(npt[:, None, :], (_T, _K, _D))
    out_ref[0] = jnp.concatenate([y_p, y_x, tiled], axis=-1)


@jax.jit
def _assemble(gp, gx, newp, stdv, ab):
    return pl.pallas_call(
        _assemble_body,
        grid=(_B, _S // _T),
        in_specs=[
            pl.BlockSpec((1, _T, _K, _D), lambda b, s: (b, s, 0, 0)),
            pl.BlockSpec((1, _T, _K, 3), lambda b, s: (b, s, 0, 0)),
            pl.BlockSpec((1, _T, _D), lambda b, s: (b, s, 0)),
            pl.BlockSpec(memory_space=pltpu.SMEM),
            pl.BlockSpec((2, _FD), lambda b, s: (0, 0)),
        ],
        out_specs=pl.BlockSpec((1, _T, _K, _FD + _D), lambda b, s: (b, s, 0, 0)),
        out_shape=jax.ShapeDtypeStruct((_B, _S, _K, _FD + _D), jnp.float32),
    )(gp, gx, newp, stdv, ab)


def kernel(xyz, points, affine_alpha, affine_beta):
    xyz_t = jnp.transpose(xyz, (2, 0, 1))
    fps_idx, nxyz = _fps(xyz_t)
    new_xyz = jnp.transpose(nxyz, (1, 2, 0))
    xyz_q = jnp.transpose(xyz, (0, 2, 1))[:, :, None, :]
    idx_g = _knn(new_xyz, xyz_q)

    boff = (jnp.arange(_B, dtype=jnp.int32) * _N)[:, None]
    all_idx = jnp.concatenate(
        [idx_g.reshape(-1), (fps_idx + boff).reshape(-1)]
    )
    rows = _sc_gather_rows(points.reshape(_B * _N, _D), all_idx)
    gp = rows[: _B * _S * _K].reshape(_B, _S, _K, _D)
    newp = rows[_B * _S * _K :].reshape(_B, _S, _D)

    idx3 = (idx_g.reshape(-1, 1) * 3 + jnp.arange(3, dtype=jnp.int32)).reshape(-1)
    gx = _sc_gather_words(xyz.reshape(_B * _N * _C), idx3).reshape(_B, _S, _K, _C)

    stats = _stats(gp, gx)
    s1 = stats[:, 0]
    s2 = stats[:, 1]
    n = _S * _K * _FD
    var = (s2 - s1 * s1 / n) / (n - 1)
    stdv = jnp.sqrt(var).reshape(_B, 1)
    ab = jnp.concatenate(
        [affine_alpha.reshape(1, _FD), affine_beta.reshape(1, _FD)]
    )
    new_points_out = _assemble(gp, gx, newp, stdv, ab)
    return (new_xyz, new_points_out)
